# baseline (speedup 1.0000x reference)
import jax
import jax.numpy as jnp
from jax import lax
from jax.experimental import pallas as pl
from jax.experimental.pallas import tpu as pltpu
from jax.experimental.pallas import tpu_sc as plsc

D = 32
B = 16384
NC, NS, L = 2, 16, 16
NW = NC * NS
BPW = B // NW
CHUNKS = BPW // L


def _rsqrt_newton(s):
    i = plsc.bitcast(s, jnp.int32)
    i = jnp.int32(0x5F3759DF) - lax.shift_right_logical(i, 1)
    y = plsc.bitcast(i, jnp.float32)
    half_s = 0.5 * s
    for _ in range(3):
        y = y * (1.5 - half_s * y * y)
    return y


def _sc_body(ent_hbm, rel_hbm, ph, pr, pt, nh, nr, nt,
             o0, o1, o2, o3, o4, o5, idx_v, land_v, out_v, sem):
    wid = lax.axis_index("s") * NC + lax.axis_index("c")
    base = wid * BPW
    jobs = ((ent_hbm, ph, o0), (rel_hbm, pr, o1), (ent_hbm, pt, o2),
            (ent_hbm, nh, o3), (rel_hbm, nr, o4), (ent_hbm, nt, o5))

    for table, idx_hbm, out_hbm in jobs:
        pltpu.sync_copy(idx_hbm.at[pl.ds(base, BPW)], idx_v)
        for j in range(D):
            pltpu.async_copy(table.at[j].at[idx_v], land_v.at[j], sem)
        pltpu.make_async_copy(table.at[:, pl.ds(0, BPW)], land_v, sem).wait()

        def norm_body(c, _):
            cols = [land_v[j, pl.ds(c * L, L)] for j in range(D)]
            s = cols[0] * cols[0]
            for j in range(1, D):
                s = s + cols[j] * cols[j]
            y = _rsqrt_newton(jnp.maximum(s, 1e-24))
            for j in range(D):
                out_v[j, pl.ds(c * L, L)] = cols[j] * y
            return _

        lax.fori_loop(0, CHUNKS, norm_body, None)
        pltpu.sync_copy(out_v, out_hbm.at[:, pl.ds(base, BPW)])


@jax.jit
def kernel(rv_ent_embeds, rel_embeds, rel_pos_hs, rel_pos_rs, rel_pos_ts,
           rel_neg_hs, rel_neg_rs, rel_neg_ts):
    ent_t = rv_ent_embeds.T
    rel_t = rel_embeds.T
    out = jax.ShapeDtypeStruct((D, B), jnp.float32)
    mesh = plsc.VectorSubcoreMesh(core_axis_name="c", subcore_axis_name="s",
                                  num_cores=NC, num_subcores=NS)
    run = pl.kernel(
        _sc_body,
        out_type=(out,) * 6,
        mesh=mesh,
        compiler_params=pltpu.CompilerParams(needs_layout_passes=False,
                                             use_tc_tiling_on_sc=False),
        scratch_types=[
            pltpu.VMEM((BPW,), jnp.int32),
            pltpu.VMEM((D, BPW), jnp.float32),
            pltpu.VMEM((D, BPW), jnp.float32),
            pltpu.SemaphoreType.DMA,
        ],
    )
    outs = run(ent_t, rel_t, rel_pos_hs, rel_pos_rs,
               rel_pos_ts, rel_neg_hs, rel_neg_rs, rel_neg_ts)
    return tuple(o.T for o in outs)


# speedup vs baseline: 7.5427x; 7.5427x over previous
import jax
import jax.numpy as jnp
from jax import lax
from jax.experimental import pallas as pl
from jax.experimental.pallas import tpu as pltpu
from jax.experimental.pallas import tpu_sc as plsc

D = 32
B = 16384
NC, NS, L = 2, 16, 16
NW = NC * NS
BPW = B // NW
CHUNKS = BPW // L


def _rsqrt_newton(s):
    i = plsc.bitcast(s, jnp.int32)
    i = jnp.int32(0x5F3759DF) - lax.shift_right_logical(i, 1)
    y = plsc.bitcast(i, jnp.float32)
    half_s = 0.5 * s
    for _ in range(3):
        y = y * (1.5 - half_s * y * y)
    return y


def _sc_body(ent_hbm, rel_hbm, ph, pr, pt, nh, nr, nt,
             o0, o1, o2, o3, o4, o5, idx_v, land_v, out_v, sem):
    wid = lax.axis_index("s") * NC + lax.axis_index("c")
    base = wid * BPW
    jobs = ((ent_hbm, ph, o0), (rel_hbm, pr, o1), (ent_hbm, pt, o2),
            (ent_hbm, nh, o3), (rel_hbm, nr, o4), (ent_hbm, nt, o5))

    lanes = lax.iota(jnp.int32, L)
    diag = [lax.bitwise_and(lanes + d, jnp.int32(D - 1)) for d in range(D)]

    for table, idx_hbm, out_hbm in jobs:
        pltpu.sync_copy(idx_hbm.at[pl.ds(base, BPW)], idx_v)

        def row_body(c, _):
            chunk = idx_v[pl.ds(c * L, L)]
            for jj in range(L):
                pltpu.async_copy(table.at[pl.ds(chunk[jj], 1)],
                                 land_v.at[pl.ds(c * L + jj, 1)], sem)
            return _

        lax.fori_loop(0, CHUNKS, row_body, None)
        pltpu.make_async_copy(table.at[pl.ds(0, BPW)], land_v, sem).wait()

        def norm_body(c, _):
            row_ids = c * L + lanes
            vals = [plsc.load_gather(land_v, [row_ids, diag[d]])
                    for d in range(D)]
            s = vals[0] * vals[0]
            for d in range(1, D):
                s = s + vals[d] * vals[d]
            y = _rsqrt_newton(jnp.maximum(s, 1e-24))
            for d in range(D):
                plsc.store_scatter(out_v, [diag[d], row_ids], vals[d] * y)
            return _

        lax.fori_loop(0, CHUNKS, norm_body, None)
        pltpu.sync_copy(out_v, out_hbm.at[:, pl.ds(base, BPW)])


@jax.jit
def kernel(rv_ent_embeds, rel_embeds, rel_pos_hs, rel_pos_rs, rel_pos_ts,
           rel_neg_hs, rel_neg_rs, rel_neg_ts):
    out = jax.ShapeDtypeStruct((D, B), jnp.float32)
    mesh = plsc.VectorSubcoreMesh(core_axis_name="c", subcore_axis_name="s",
                                  num_cores=NC, num_subcores=NS)
    run = pl.kernel(
        _sc_body,
        out_type=(out,) * 6,
        mesh=mesh,
        compiler_params=pltpu.CompilerParams(needs_layout_passes=False,
                                             use_tc_tiling_on_sc=True),
        scratch_types=[
            pltpu.VMEM((BPW,), jnp.int32),
            pltpu.VMEM((BPW, D), jnp.float32),
            pltpu.VMEM((D, BPW), jnp.float32),
            pltpu.SemaphoreType.DMA,
        ],
    )
    outs = run(rv_ent_embeds, rel_embeds, rel_pos_hs, rel_pos_rs,
               rel_pos_ts, rel_neg_hs, rel_neg_rs, rel_neg_ts)
    return tuple(o.T for o in outs)
